# baseline (speedup 1.0000x reference)
import functools

import jax
import jax.numpy as jnp
from jax import lax
from jax.experimental import pallas as pl
from jax.experimental.pallas import tpu as pltpu
from jax.experimental.pallas import tpu_sc as plsc


def kernel(images_batch, masks_batch):
    del masks_batch
    B, ve_dim, feature_dim = images_batch.shape
    rows = B * ve_dim
    flat = images_batch.reshape(rows, feature_dim)

    info = plsc.get_sparse_core_info()
    nw = info.num_cores * info.num_subcores
    rows_per_worker = rows // nw
    mesh = plsc.VectorSubcoreMesh(core_axis_name="c", subcore_axis_name="s")

    @functools.partial(
        pl.kernel,
        out_type=jax.ShapeDtypeStruct((rows, feature_dim), flat.dtype),
        mesh=mesh,
    )
    def sc_copy(in_hbm, out_hbm):
        wid = lax.axis_index("s") * info.num_cores + lax.axis_index("c")
        base = wid * rows_per_worker
        pltpu.sync_copy(
            in_hbm.at[pl.ds(base, rows_per_worker)],
            out_hbm.at[pl.ds(base, rows_per_worker)],
        )

    return sc_copy(flat).reshape(B, ve_dim, feature_dim)


# speedup vs baseline: 28.0623x; 28.0623x over previous
import functools

import jax
import jax.numpy as jnp
from jax import lax
from jax.experimental import pallas as pl
from jax.experimental.pallas import tpu as pltpu
from jax.experimental.pallas import tpu_sc as plsc

_CHUNK_ROWS = 64


def kernel(images_batch, masks_batch):
    del masks_batch
    B, ve_dim, feature_dim = images_batch.shape
    rows = B * ve_dim
    flat = images_batch.reshape(rows, feature_dim)

    info = plsc.get_sparse_core_info()
    nw = info.num_cores * info.num_subcores
    rpw = rows // nw
    ch = _CHUNK_ROWS
    nchunks = rpw // ch
    mesh = plsc.VectorSubcoreMesh(core_axis_name="c", subcore_axis_name="s")

    @functools.partial(
        pl.kernel,
        out_type=jax.ShapeDtypeStruct((rows, feature_dim), flat.dtype),
        mesh=mesh,
        scratch_types=[
            pltpu.VMEM((ch, feature_dim), jnp.float32),
            pltpu.VMEM((ch, feature_dim), jnp.float32),
            pltpu.SemaphoreType.DMA,
            pltpu.SemaphoreType.DMA,
            pltpu.SemaphoreType.DMA,
            pltpu.SemaphoreType.DMA,
        ],
    )
    def sc_copy(in_hbm, out_hbm, buf0, buf1, gi0, gi1, so0, so1):
        wid = lax.axis_index("s") * info.num_cores + lax.axis_index("c")
        base = wid * rpw
        bufs = (buf0, buf1)
        gsems = (gi0, gi1)
        ssems = (so0, so1)

        def gather(i):
            b = i % 2
            return pltpu.make_async_copy(
                in_hbm.at[pl.ds(base + i * ch, ch)], bufs[b], gsems[b])

        def scatter(i):
            b = i % 2
            return pltpu.make_async_copy(
                bufs[b], out_hbm.at[pl.ds(base + i * ch, ch)], ssems[b])

        gather(0).start()
        for i in range(nchunks):
            gather(i).wait()
            scatter(i).start()
            if i + 1 < nchunks:
                if i >= 1:
                    scatter(i - 1).wait()
                gather(i + 1).start()
        scatter(nchunks - 1).wait()

    return sc_copy(flat).reshape(B, ve_dim, feature_dim)
